# baseline (speedup 1.0000x reference)
import jax
import jax.numpy as jnp
import numpy as np
from jax import lax
from jax.experimental import pallas as pl
from jax.experimental.pallas import tpu as pltpu
from jax.experimental.pallas import tpu_sc as plsc

_B = 16384
_K = 16
_DM = 32
_NC = 2
_NS = 16
_NW = _NC * _NS
_BPW = _B // _NW
_NJ = 8
_CB = _BPW // _NJ

_OW0 = 0
_OW1 = _OW0 + 32 * 64
_OWS = _OW1 + 16 * 32
_OB0 = _OWS + 32
_OB1 = _OB0 + 32
_OBS = _OB1 + 16
_WPACK = 2656


def _body(x_hbm, gidx_hbm, umf_hbm, umlp_hbm, imf_hbm, imlp_hbm, wpack_hbm,
          out_hbm, xv, ixumf, ixuml, iximf, iximl, gumf, guml, gimf, giml,
          wv, outv, sem0, sem1):
    wid = lax.axis_index("s") * _NC + lax.axis_index("c")
    base = wid * _BPW
    iota = lax.iota(jnp.int32, 16)

    def splat(off):
        return plsc.load_gather(wv, [jnp.full((16,), off, jnp.int32)])

    pltpu.sync_copy(x_hbm.at[pl.ds(base * 2, _BPW * 2)], xv)
    pltpu.sync_copy(gidx_hbm.at[0, wid], ixumf)
    pltpu.sync_copy(gidx_hbm.at[1, wid], ixuml)
    pltpu.sync_copy(gidx_hbm.at[2, wid], iximf)
    pltpu.sync_copy(gidx_hbm.at[3, wid], iximl)
    pltpu.sync_copy(wpack_hbm, wv)

    def fire(j, p):
        sem = sem0 if p == 0 else sem1
        return [
            pltpu.async_copy(umf_hbm.at[ixumf.at[j]], gumf.at[p], sem),
            pltpu.async_copy(umlp_hbm.at[ixuml.at[j]], guml.at[p], sem),
            pltpu.async_copy(imf_hbm.at[iximf.at[j]], gimf.at[p], sem),
            pltpu.async_copy(imlp_hbm.at[iximl.at[j]], giml.at[p], sem),
        ]

    u = jnp.zeros((16,), jnp.float32)
    for k in range(32):
        col = plsc.load_gather(wv, [_OW1 + iota * 32 + k])
        u = u + splat(_OB0 + k) * col
    w_mlp = wv[pl.ds(_OWS + 16, 16)]
    b1v = wv[pl.ds(_OB1, 16)]
    c_base = jnp.sum(w_mlp * (u + b1v)) + splat(_OBS)

    t0 = jnp.zeros((16,), jnp.float32)
    t1 = jnp.zeros((16,), jnp.float32)
    for k in range(16):
        wk = splat(_OWS + 16 + k)
        t0 = t0 + wk * wv[pl.ds(_OW1 + k * 32, 16)]
        t1 = t1 + wk * wv[pl.ds(_OW1 + k * 32 + 16, 16)]

    vq = [jnp.zeros((16,), jnp.float32) for _ in range(4)]
    for j in range(32):
        oh = (iota == (j % 16)).astype(jnp.float32)
        tj = jnp.sum((t0 if j < 16 else t1) * oh)
        for q in range(4):
            vq[q] = vq[q] + tj * wv[pl.ds(_OW0 + j * 64 + q * 16, 16)]
    su = []
    si = []
    for k in range(_DM):
        oh = (iota == (k % 16)).astype(jnp.float32)
        su.append(jnp.sum(vq[k // 16] * oh))
        si.append(jnp.sum(vq[2 + k // 16] * oh))

    def consume(j, p):
        psplat = jnp.full((16,), p, jnp.int32)
        for sub in range(_CB // 16):
            rows = j * _CB + sub * 16 + iota
            uid = plsc.load_gather(xv, [rows * 2])
            iid = plsc.load_gather(xv, [rows * 2 + 1])
            dlanes = sub * 16 + iota
            cumf = (uid & 7) * 16
            cuml = (uid & 3) * 32
            cimf = (iid & 7) * 16
            ciml = (iid & 3) * 32
            acc = c_base
            for k in range(_K):
                a = plsc.load_gather(gumf, [psplat, dlanes, cumf + k])
                b = plsc.load_gather(gimf, [psplat, dlanes, cimf + k])
                acc = acc + a * b * splat(_OWS + k)
            for k in range(_DM):
                a = plsc.load_gather(guml, [psplat, dlanes, cuml + k])
                b = plsc.load_gather(giml, [psplat, dlanes, ciml + k])
                acc = acc + a * su[k] + b * si[k]
            outv[pl.ds(j * _CB + sub * 16, 16)] = acc

    def step(g, carry):
        j0 = g * 2
        d0 = fire(j0, 0)
        for d in d0:
            d.wait()
        consume(j0, 0)
        d1 = fire(j0 + 1, 1)
        for d in d1:
            d.wait()
        consume(j0 + 1, 1)
        return carry

    lax.fori_loop(0, _NJ // 2, step, 0)
    pltpu.sync_copy(outv, out_hbm.at[pl.ds(base, _BPW)])


@jax.jit
def _ncf_sc(xflat, gidx, umf, umlp, imf, imlp, wpack):
    mesh = plsc.VectorSubcoreMesh(core_axis_name="c", subcore_axis_name="s")
    return pl.kernel(
        _body,
        out_type=jax.ShapeDtypeStruct((_B,), jnp.float32),
        mesh=mesh,
        compiler_params=pltpu.CompilerParams(needs_layout_passes=False),
        scratch_types=[
            pltpu.VMEM((_BPW * 2,), jnp.int32),
            pltpu.VMEM((_NJ, _CB), jnp.int32),
            pltpu.VMEM((_NJ, _CB), jnp.int32),
            pltpu.VMEM((_NJ, _CB), jnp.int32),
            pltpu.VMEM((_NJ, _CB), jnp.int32),
            pltpu.VMEM((2, _CB, 128), jnp.float32),
            pltpu.VMEM((2, _CB, 128), jnp.float32),
            pltpu.VMEM((2, _CB, 128), jnp.float32),
            pltpu.VMEM((2, _CB, 128), jnp.float32),
            pltpu.VMEM((_WPACK,), jnp.float32),
            pltpu.VMEM((_BPW,), jnp.float32),
            pltpu.SemaphoreType.DMA,
            pltpu.SemaphoreType.DMA,
        ],
    )(xflat, gidx, umf, umlp, imf, imlp, wpack)


_TRB = 2048


def _tr_body(umf_t, umlp_t, imf_t, imlp_t, umf_o, umlp_o, imf_o, imlp_o):
    def tr(o_ref, x_ref, d):
        eye = jnp.eye(d, dtype=jnp.float32)
        o_ref[...] = jax.lax.dot_general(
            x_ref[...], eye, (((0,), (0,)), ((), ())),
            preferred_element_type=jnp.float32)

    tr(umf_o, umf_t, 16)
    tr(umlp_o, umlp_t, 32)
    tr(imf_o, imf_t, 16)
    tr(imlp_o, imlp_t, 32)


@jax.jit
def _transpose_tables(umf_t, umlp_t, imf_t, imlp_t):
    n = umf_t.shape[1]
    grid = (n + _TRB - 1) // _TRB
    return pl.pallas_call(
        _tr_body,
        grid=(grid,),
        in_specs=[
            pl.BlockSpec((16, _TRB), lambda i: (0, i)),
            pl.BlockSpec((32, _TRB), lambda i: (0, i)),
            pl.BlockSpec((16, _TRB), lambda i: (0, i)),
            pl.BlockSpec((32, _TRB), lambda i: (0, i)),
        ],
        out_specs=[
            pl.BlockSpec((_TRB, 16), lambda i: (i, 0)),
            pl.BlockSpec((_TRB, 32), lambda i: (i, 0)),
            pl.BlockSpec((_TRB, 16), lambda i: (i, 0)),
            pl.BlockSpec((_TRB, 32), lambda i: (i, 0)),
        ],
        out_shape=[
            jax.ShapeDtypeStruct((n, 16), jnp.float32),
            jax.ShapeDtypeStruct((n, 32), jnp.float32),
            jax.ShapeDtypeStruct((n, 16), jnp.float32),
            jax.ShapeDtypeStruct((n, 32), jnp.float32),
        ],
        compiler_params=pltpu.CompilerParams(
            dimension_semantics=("arbitrary",)),
    )(umf_t, umlp_t, imf_t, imlp_t)


def kernel(x, emb_u_mf, emb_u_mlp, emb_i_mf, emb_i_mlp, W0, b0, W1, b1, Ws, bs):
    wpack = jnp.concatenate([
        W0.ravel(), W1.ravel(), Ws.ravel(), b0, b1, bs,
        jnp.zeros((_WPACK - _OBS - 1,), jnp.float32),
    ])
    xi = x.astype(jnp.int32)
    uid, iid = xi[:, 0], xi[:, 1]
    gidx = jnp.stack([uid >> 3, uid >> 2, iid >> 3, iid >> 2]).reshape(
        4, _NW, _NJ, _CB)
    umf, umlp, imf, imlp = _transpose_tables(
        emb_u_mf.T, emb_u_mlp.T, emb_i_mf.T, emb_i_mlp.T)

    out = _ncf_sc(
        xi.reshape(-1),
        gidx,
        umf.reshape(-1, 128),
        umlp.reshape(-1, 128),
        imf.reshape(-1, 128),
        imlp.reshape(-1, 128),
        wpack,
    )
    return out.reshape(_B, 1)


# speedup vs baseline: 1.0937x; 1.0937x over previous
import jax
import jax.numpy as jnp
import numpy as np
from jax import lax
from jax.experimental import pallas as pl
from jax.experimental.pallas import tpu as pltpu
from jax.experimental.pallas import tpu_sc as plsc

_B = 16384
_K = 16
_DM = 32
_NC = 2
_NS = 16
_NW = _NC * _NS
_BPW = _B // _NW
_NJ = 8
_CB = _BPW // _NJ

_OW0 = 0
_OW1 = _OW0 + 32 * 64
_OWS = _OW1 + 16 * 32
_OB0 = _OWS + 32
_OB1 = _OB0 + 32
_OBS = _OB1 + 16
_WPACK = 2656


def _body(x_hbm, gidx_hbm, umf_hbm, umlp_hbm, imf_hbm, imlp_hbm, wpack_hbm,
          out_hbm, xv, ixumf, ixuml, iximf, iximl, gumf, guml, gimf, giml,
          wv, outv, sem0, sem1):
    wid = lax.axis_index("s") * _NC + lax.axis_index("c")
    base = wid * _BPW
    iota = lax.iota(jnp.int32, 16)

    def splat(off):
        return plsc.load_gather(wv, [jnp.full((16,), off, jnp.int32)])

    pltpu.sync_copy(x_hbm.at[pl.ds(base * 2, _BPW * 2)], xv)
    pltpu.sync_copy(gidx_hbm.at[0, wid], ixumf)
    pltpu.sync_copy(gidx_hbm.at[1, wid], ixuml)
    pltpu.sync_copy(gidx_hbm.at[2, wid], iximf)
    pltpu.sync_copy(gidx_hbm.at[3, wid], iximl)
    pltpu.sync_copy(wpack_hbm, wv)

    def fire(j, p):
        sem = sem0 if p == 0 else sem1
        return [
            pltpu.async_copy(umf_hbm.at[ixumf.at[j]], gumf.at[p], sem),
            pltpu.async_copy(umlp_hbm.at[ixuml.at[j]], guml.at[p], sem),
            pltpu.async_copy(imf_hbm.at[iximf.at[j]], gimf.at[p], sem),
            pltpu.async_copy(imlp_hbm.at[iximl.at[j]], giml.at[p], sem),
        ]

    u = jnp.zeros((16,), jnp.float32)
    for k in range(32):
        col = plsc.load_gather(wv, [_OW1 + iota * 32 + k])
        u = u + splat(_OB0 + k) * col
    w_mlp = wv[pl.ds(_OWS + 16, 16)]
    b1v = wv[pl.ds(_OB1, 16)]
    c_base = jnp.sum(w_mlp * (u + b1v)) + splat(_OBS)

    t0 = jnp.zeros((16,), jnp.float32)
    t1 = jnp.zeros((16,), jnp.float32)
    for k in range(16):
        wk = splat(_OWS + 16 + k)
        t0 = t0 + wk * wv[pl.ds(_OW1 + k * 32, 16)]
        t1 = t1 + wk * wv[pl.ds(_OW1 + k * 32 + 16, 16)]

    vq = [jnp.zeros((16,), jnp.float32) for _ in range(4)]
    for j in range(32):
        oh = (iota == (j % 16)).astype(jnp.float32)
        tj = jnp.sum((t0 if j < 16 else t1) * oh)
        for q in range(4):
            vq[q] = vq[q] + tj * wv[pl.ds(_OW0 + j * 64 + q * 16, 16)]
    su = []
    si = []
    for k in range(_DM):
        oh = (iota == (k % 16)).astype(jnp.float32)
        su.append(jnp.sum(vq[k // 16] * oh))
        si.append(jnp.sum(vq[2 + k // 16] * oh))

    def consume(j, p):
        psplat = jnp.full((16,), p, jnp.int32)
        for sub in range(_CB // 16):
            rows = j * _CB + sub * 16 + iota
            uid = plsc.load_gather(xv, [rows * 2])
            iid = plsc.load_gather(xv, [rows * 2 + 1])
            dlanes = sub * 16 + iota
            cumf = (uid & 7) * 16
            cuml = (uid & 3) * 32
            cimf = (iid & 7) * 16
            ciml = (iid & 3) * 32
            acc = c_base
            for k in range(_K):
                a = plsc.load_gather(gumf, [psplat, dlanes, cumf + k])
                b = plsc.load_gather(gimf, [psplat, dlanes, cimf + k])
                acc = acc + a * b * splat(_OWS + k)
            for k in range(_DM):
                a = plsc.load_gather(guml, [psplat, dlanes, cuml + k])
                b = plsc.load_gather(giml, [psplat, dlanes, ciml + k])
                acc = acc + a * su[k] + b * si[k]
            outv[pl.ds(j * _CB + sub * 16, 16)] = acc

    def step(g, carry):
        j0 = g * 2
        d0 = fire(j0, 0)
        for d in d0:
            d.wait()
        consume(j0, 0)
        d1 = fire(j0 + 1, 1)
        for d in d1:
            d.wait()
        consume(j0 + 1, 1)
        return carry

    lax.fori_loop(0, _NJ // 2, step, 0)
    pltpu.sync_copy(outv, out_hbm.at[pl.ds(base, _BPW)])


@jax.jit
def _ncf_sc(xflat, gidx, umf, umlp, imf, imlp, wpack):
    mesh = plsc.VectorSubcoreMesh(core_axis_name="c", subcore_axis_name="s")
    return pl.kernel(
        _body,
        out_type=jax.ShapeDtypeStruct((_B,), jnp.float32),
        mesh=mesh,
        compiler_params=pltpu.CompilerParams(needs_layout_passes=False),
        scratch_types=[
            pltpu.VMEM((_BPW * 2,), jnp.int32),
            pltpu.VMEM((_NJ, _CB), jnp.int32),
            pltpu.VMEM((_NJ, _CB), jnp.int32),
            pltpu.VMEM((_NJ, _CB), jnp.int32),
            pltpu.VMEM((_NJ, _CB), jnp.int32),
            pltpu.VMEM((2, _CB, 128), jnp.float32),
            pltpu.VMEM((2, _CB, 128), jnp.float32),
            pltpu.VMEM((2, _CB, 128), jnp.float32),
            pltpu.VMEM((2, _CB, 128), jnp.float32),
            pltpu.VMEM((_WPACK,), jnp.float32),
            pltpu.VMEM((_BPW,), jnp.float32),
            pltpu.SemaphoreType.DMA,
            pltpu.SemaphoreType.DMA,
        ],
    )(xflat, gidx, umf, umlp, imf, imlp, wpack)


_TRB = 8192


def _tr_body(umf_t, umlp_t, imf_t, imlp_t, umf_o, umlp_o, imf_o, imlp_o):
    def tr(o_ref, x_ref, d):
        eye = jnp.eye(d, dtype=jnp.float32)
        o_ref[...] = jax.lax.dot_general(
            x_ref[...], eye, (((0,), (0,)), ((), ())),
            preferred_element_type=jnp.float32)

    tr(umf_o, umf_t, 16)
    tr(umlp_o, umlp_t, 32)
    tr(imf_o, imf_t, 16)
    tr(imlp_o, imlp_t, 32)


@jax.jit
def _transpose_tables(umf_t, umlp_t, imf_t, imlp_t):
    n = umf_t.shape[1]
    grid = (n + _TRB - 1) // _TRB
    return pl.pallas_call(
        _tr_body,
        grid=(grid,),
        in_specs=[
            pl.BlockSpec((16, _TRB), lambda i: (0, i)),
            pl.BlockSpec((32, _TRB), lambda i: (0, i)),
            pl.BlockSpec((16, _TRB), lambda i: (0, i)),
            pl.BlockSpec((32, _TRB), lambda i: (0, i)),
        ],
        out_specs=[
            pl.BlockSpec((_TRB, 16), lambda i: (i, 0)),
            pl.BlockSpec((_TRB, 32), lambda i: (i, 0)),
            pl.BlockSpec((_TRB, 16), lambda i: (i, 0)),
            pl.BlockSpec((_TRB, 32), lambda i: (i, 0)),
        ],
        out_shape=[
            jax.ShapeDtypeStruct((n, 16), jnp.float32),
            jax.ShapeDtypeStruct((n, 32), jnp.float32),
            jax.ShapeDtypeStruct((n, 16), jnp.float32),
            jax.ShapeDtypeStruct((n, 32), jnp.float32),
        ],
        compiler_params=pltpu.CompilerParams(
            dimension_semantics=("arbitrary",)),
    )(umf_t, umlp_t, imf_t, imlp_t)


def kernel(x, emb_u_mf, emb_u_mlp, emb_i_mf, emb_i_mlp, W0, b0, W1, b1, Ws, bs):
    wpack = jnp.concatenate([
        W0.ravel(), W1.ravel(), Ws.ravel(), b0, b1, bs,
        jnp.zeros((_WPACK - _OBS - 1,), jnp.float32),
    ])
    xi = x.astype(jnp.int32)
    uid, iid = xi[:, 0], xi[:, 1]
    gidx = jnp.stack([uid >> 3, uid >> 2, iid >> 3, iid >> 2]).reshape(
        4, _NW, _NJ, _CB)
    umf, umlp, imf, imlp = _transpose_tables(
        emb_u_mf.T, emb_u_mlp.T, emb_i_mf.T, emb_i_mlp.T)

    out = _ncf_sc(
        xi.reshape(-1),
        gidx,
        umf.reshape(-1, 128),
        umlp.reshape(-1, 128),
        imf.reshape(-1, 128),
        imlp.reshape(-1, 128),
        wpack,
    )
    return out.reshape(_B, 1)


# speedup vs baseline: 1.4313x; 1.3087x over previous
import jax
import jax.numpy as jnp
import numpy as np
from jax import lax
from jax.experimental import pallas as pl
from jax.experimental.pallas import tpu as pltpu
from jax.experimental.pallas import tpu_sc as plsc

_B = 16384
_K = 16
_DM = 32
_NC = 2
_NS = 16
_NW = _NC * _NS
_BPW = _B // _NW
_NJ = 8
_CB = _BPW // _NJ

_OW0 = 0
_OW1 = _OW0 + 32 * 64
_OWS = _OW1 + 16 * 32
_OB0 = _OWS + 32
_OB1 = _OB0 + 32
_OBS = _OB1 + 16
_WPACK = 2656


def _body(x_hbm, gidx_hbm, umf_hbm, umlp_hbm, imf_hbm, imlp_hbm, wpack_hbm,
          out_hbm, xv, ixumf, ixuml, iximf, iximl, gumf, guml, gimf, giml,
          wv, outv, sem0, sem1):
    wid = lax.axis_index("s") * _NC + lax.axis_index("c")
    base = wid * _BPW
    iota = lax.iota(jnp.int32, 16)

    def splat(off):
        return plsc.load_gather(wv, [jnp.full((16,), off, jnp.int32)])

    pltpu.sync_copy(x_hbm.at[pl.ds(base * 2, _BPW * 2)], xv)
    pltpu.sync_copy(gidx_hbm.at[0, wid], ixumf)
    pltpu.sync_copy(gidx_hbm.at[1, wid], ixuml)
    pltpu.sync_copy(gidx_hbm.at[2, wid], iximf)
    pltpu.sync_copy(gidx_hbm.at[3, wid], iximl)
    pltpu.sync_copy(wpack_hbm, wv)

    def fire(j, p):
        sem = sem0 if p == 0 else sem1
        return [
            pltpu.async_copy(umf_hbm.at[ixumf.at[j]], gumf.at[p], sem),
            pltpu.async_copy(umlp_hbm.at[ixuml.at[j]], guml.at[p], sem),
            pltpu.async_copy(imf_hbm.at[iximf.at[j]], gimf.at[p], sem),
            pltpu.async_copy(imlp_hbm.at[iximl.at[j]], giml.at[p], sem),
        ]

    u = jnp.zeros((16,), jnp.float32)
    for k in range(32):
        col = plsc.load_gather(wv, [_OW1 + iota * 32 + k])
        u = u + splat(_OB0 + k) * col
    w_mlp = wv[pl.ds(_OWS + 16, 16)]
    b1v = wv[pl.ds(_OB1, 16)]
    c_base = jnp.sum(w_mlp * (u + b1v)) + splat(_OBS)

    t0 = jnp.zeros((16,), jnp.float32)
    t1 = jnp.zeros((16,), jnp.float32)
    for k in range(16):
        wk = splat(_OWS + 16 + k)
        t0 = t0 + wk * wv[pl.ds(_OW1 + k * 32, 16)]
        t1 = t1 + wk * wv[pl.ds(_OW1 + k * 32 + 16, 16)]

    vq = [jnp.zeros((16,), jnp.float32) for _ in range(4)]
    for j in range(32):
        oh = (iota == (j % 16)).astype(jnp.float32)
        tj = jnp.sum((t0 if j < 16 else t1) * oh)
        for q in range(4):
            vq[q] = vq[q] + tj * wv[pl.ds(_OW0 + j * 64 + q * 16, 16)]
    su = []
    si = []
    for k in range(_DM):
        oh = (iota == (k % 16)).astype(jnp.float32)
        su.append(jnp.sum(vq[k // 16] * oh))
        si.append(jnp.sum(vq[2 + k // 16] * oh))

    def consume(j, p):
        psplat = jnp.full((16,), p, jnp.int32)
        for sub in range(_CB // 16):
            rows = j * _CB + sub * 16 + iota
            uid = plsc.load_gather(xv, [rows * 2])
            iid = plsc.load_gather(xv, [rows * 2 + 1])
            dlanes = sub * 16 + iota
            cumf = (uid & 7) * 16
            cuml = (uid & 3) * 32
            cimf = (iid & 7) * 16
            ciml = (iid & 3) * 32
            acc = c_base
            for k in range(_K):
                a = plsc.load_gather(gumf, [psplat, dlanes, cumf + k])
                b = plsc.load_gather(gimf, [psplat, dlanes, cimf + k])
                acc = acc + a * b * splat(_OWS + k)
            for k in range(_DM):
                a = plsc.load_gather(guml, [psplat, dlanes, cuml + k])
                b = plsc.load_gather(giml, [psplat, dlanes, ciml + k])
                acc = acc + a * su[k] + b * si[k]
            outv[pl.ds(j * _CB + sub * 16, 16)] = acc

    def step(g, carry):
        j0 = g * 2
        d0 = fire(j0, 0)
        for d in d0:
            d.wait()
        consume(j0, 0)
        d1 = fire(j0 + 1, 1)
        for d in d1:
            d.wait()
        consume(j0 + 1, 1)
        return carry

    lax.fori_loop(0, _NJ // 2, step, 0)
    pltpu.sync_copy(outv, out_hbm.at[pl.ds(base, _BPW)])


@jax.jit
def _ncf_sc(xflat, gidx, umf, umlp, imf, imlp, wpack):
    mesh = plsc.VectorSubcoreMesh(core_axis_name="c", subcore_axis_name="s")
    return pl.kernel(
        _body,
        out_type=jax.ShapeDtypeStruct((_B,), jnp.float32),
        mesh=mesh,
        compiler_params=pltpu.CompilerParams(needs_layout_passes=False),
        scratch_types=[
            pltpu.VMEM((_BPW * 2,), jnp.int32),
            pltpu.VMEM((_NJ, _CB), jnp.int32),
            pltpu.VMEM((_NJ, _CB), jnp.int32),
            pltpu.VMEM((_NJ, _CB), jnp.int32),
            pltpu.VMEM((_NJ, _CB), jnp.int32),
            pltpu.VMEM((2, _CB, 128), jnp.float32),
            pltpu.VMEM((2, _CB, 128), jnp.float32),
            pltpu.VMEM((2, _CB, 128), jnp.float32),
            pltpu.VMEM((2, _CB, 128), jnp.float32),
            pltpu.VMEM((_WPACK,), jnp.float32),
            pltpu.VMEM((_BPW,), jnp.float32),
            pltpu.SemaphoreType.DMA,
            pltpu.SemaphoreType.DMA,
        ],
    )(xflat, gidx, umf, umlp, imf, imlp, wpack)


def kernel(x, emb_u_mf, emb_u_mlp, emb_i_mf, emb_i_mlp, W0, b0, W1, b1, Ws, bs):
    wpack = jnp.concatenate([
        W0.ravel(), W1.ravel(), Ws.ravel(), b0, b1, bs,
        jnp.zeros((_WPACK - _OBS - 1,), jnp.float32),
    ])
    xi = x.astype(jnp.int32)
    uid, iid = xi[:, 0], xi[:, 1]
    gidx = jnp.stack([uid >> 3, uid >> 2, iid >> 3, iid >> 2]).reshape(
        4, _NW, _NJ, _CB)
    out = _ncf_sc(
        xi.reshape(-1),
        gidx,
        emb_u_mf.reshape(-1, 128),
        emb_u_mlp.reshape(-1, 128),
        emb_i_mf.reshape(-1, 128),
        emb_i_mlp.reshape(-1, 128),
        wpack,
    )
    return out.reshape(_B, 1)
